# baseline (speedup 1.0000x reference)
import jax
import jax.numpy as jnp
from jax.experimental import pallas as pl

_K = 1024
_D = 64
_BN = 256
_KC = 128


def _vq_one(x, e_ref):
    x_sq = jnp.sum(x * x, axis=1, keepdims=True)
    minval = None
    minidx = None
    for kc in range(_K // _KC):
        emb_c = e_ref[kc * _KC:(kc + 1) * _KC, :]
        mm = jax.lax.dot_general(
            x, emb_c, (((1,), (1,)), ((), ())),
            preferred_element_type=jnp.float32)
        emb_sq = jnp.sum(emb_c * emb_c, axis=1)
        dist = (emb_sq[None, :] - 2.0 * mm) + x_sq
        cmin = jnp.min(dist, axis=1, keepdims=True)
        iota = jax.lax.broadcasted_iota(jnp.int32, dist.shape, 1)
        cidx = jnp.min(jnp.where(dist == cmin, iota + kc * _KC, _K),
                       axis=1, keepdims=True)
        if minval is None:
            minval, minidx = cmin, cidx
        else:
            upd = cmin < minval
            minval = jnp.where(upd, cmin, minval)
            minidx = jnp.where(upd, cidx, minidx)
    picked = jnp.zeros((x.shape[0], _D), jnp.float32)
    for kc in range(_K // _KC):
        emb_c = e_ref[kc * _KC:(kc + 1) * _KC, :]
        iota = jax.lax.broadcasted_iota(
            jnp.int32, (x.shape[0], _KC), 1) + kc * _KC
        onehot = (iota == minidx).astype(jnp.float32)
        picked = picked + jax.lax.dot_general(
            onehot, emb_c, (((1,), (0,)), ((), ())),
            preferred_element_type=jnp.float32)
    return picked


def _vq2_body(x1_ref, x0_ref, e1_ref, e0_ref, out_ref):
    out_ref[:, :_D] = _vq_one(x1_ref[...], e1_ref)
    out_ref[:, _D:] = _vq_one(x0_ref[...], e0_ref)


def kernel(enc0, enc1, codebook0, codebook1):
    B, T, d = enc0.shape
    n = B * T
    flat1 = enc1.reshape(n, d)
    flat0 = enc0.reshape(n, d)
    out = pl.pallas_call(
        _vq2_body,
        grid=(n // _BN,),
        in_specs=[
            pl.BlockSpec((_BN, d), lambda i: (i, 0)),
            pl.BlockSpec((_BN, d), lambda i: (i, 0)),
            pl.BlockSpec((_K, d), lambda i: (0, 0)),
            pl.BlockSpec((_K, d), lambda i: (0, 0)),
        ],
        out_specs=pl.BlockSpec((_BN, 2 * d), lambda i: (i, 0)),
        out_shape=jax.ShapeDtypeStruct((n, 2 * d), jnp.float32),
    )(flat1, flat0, codebook1, codebook0)
    return out.reshape(B, T, 2 * d)


# speedup vs baseline: 1.0027x; 1.0027x over previous
import functools

import jax
import jax.numpy as jnp
from jax import lax
from jax.experimental import pallas as pl
from jax.experimental.pallas import tpu as pltpu
from jax.experimental.pallas import tpu_sc as plsc

_K = 1024
_D = 64
_BN = 256
_KC = 128

_NC = 2
_NS = 16
_NW = _NC * _NS
_GC = 512
_DP = 128


def _argmin_one(x, e_ref, base):
    x_sq = jnp.sum(x * x, axis=1, keepdims=True)
    minval = None
    minidx = None
    for kc in range(_K // _KC):
        emb_c = e_ref[kc * _KC:(kc + 1) * _KC, :]
        mm = jax.lax.dot_general(
            x, emb_c, (((1,), (1,)), ((), ())),
            preferred_element_type=jnp.float32)
        emb_sq = jnp.sum(emb_c * emb_c, axis=1)
        dist = (emb_sq[None, :] - 2.0 * mm) + x_sq
        cmin = jnp.min(dist, axis=1, keepdims=True)
        iota = jax.lax.broadcasted_iota(jnp.int32, dist.shape, 1)
        cidx = jnp.min(jnp.where(dist == cmin, iota + kc * _KC, _K),
                       axis=1, keepdims=True)
        if minval is None:
            minval, minidx = cmin, cidx
        else:
            upd = cmin < minval
            minval = jnp.where(upd, cmin, minval)
            minidx = jnp.where(upd, cidx, minidx)
    return minidx + base


def _idx_body(x1_ref, x0_ref, e1_ref, e0_ref, i1_ref, i0_ref):
    i1_ref[0, 0, :] = _argmin_one(x1_ref[...], e1_ref, 0)[:, 0]
    i0_ref[0, 0, :] = _argmin_one(x0_ref[...], e0_ref, _K)[:, 0]


def _tc_indices(flat1, flat0, codebook1, codebook0, n):
    nb = n // _BN
    return pl.pallas_call(
        _idx_body,
        grid=(nb,),
        in_specs=[
            pl.BlockSpec((_BN, _D), lambda i: (i, 0)),
            pl.BlockSpec((_BN, _D), lambda i: (i, 0)),
            pl.BlockSpec((_K, _D), lambda i: (0, 0)),
            pl.BlockSpec((_K, _D), lambda i: (0, 0)),
        ],
        out_specs=[
            pl.BlockSpec((1, 1, _BN), lambda i: (i, 0, 0)),
            pl.BlockSpec((1, 1, _BN), lambda i: (i, 0, 0)),
        ],
        out_shape=[
            jax.ShapeDtypeStruct((nb, 1, _BN), jnp.int32),
            jax.ShapeDtypeStruct((nb, 1, _BN), jnp.int32),
        ],
    )(flat1, flat0, codebook1, codebook0)


def _sc_gather(table, idx_all, total):
    gw = total // _NW
    mesh = plsc.VectorSubcoreMesh(core_axis_name="c", subcore_axis_name="s")

    @functools.partial(
        pl.kernel, mesh=mesh,
        out_type=jax.ShapeDtypeStruct((total, _DP), jnp.float32),
        scratch_types=[
            pltpu.VMEM((_GC,), jnp.int32),
            pltpu.VMEM((_GC, _DP), jnp.float32),
            pltpu.SemaphoreType.DMA,
        ],
    )
    def gather_k(table_hbm, idx_hbm, out_hbm, idx_v, rows_v, sem):
        wid = lax.axis_index("s") * _NC + lax.axis_index("c")
        base = wid * gw
        for j in range(gw // _GC):
            off = base + j * _GC
            pltpu.sync_copy(idx_hbm.at[pl.ds(off, _GC)], idx_v)
            pltpu.async_copy(table_hbm.at[idx_v], rows_v, sem).wait()
            pltpu.sync_copy(rows_v, out_hbm.at[pl.ds(off, _GC)])

    return gather_k(table, idx_all)


def kernel(enc0, enc1, codebook0, codebook1):
    B, T, d = enc0.shape
    n = B * T
    flat1 = enc1.reshape(n, d)
    flat0 = enc0.reshape(n, d)
    idx1, idx0 = _tc_indices(flat1, flat0, codebook1, codebook0, n)
    idx_all = jnp.concatenate([idx1.reshape(n), idx0.reshape(n)])
    table = jnp.pad(jnp.concatenate([codebook1, codebook0], axis=0),
                    ((0, 0), (0, _DP - _D)))
    rows = _sc_gather(table, idx_all, 2 * n)
    return jnp.concatenate(
        [rows[:n, :_D].reshape(B, T, d), rows[n:, :_D].reshape(B, T, d)],
        axis=-1)
